# Initial kernel scaffold; baseline (speedup 1.0000x reference)
import jax
import jax.numpy as jnp
from jax.experimental import pallas as pl


def kernel(x, edge_index, batch, edge_weight, params):
    raise NotImplementedError("write your pallas kernel here")


# speedup vs baseline: 1.0330x; 1.0330x over previous
import functools

import jax
import jax.numpy as jnp
from jax import lax
from jax.experimental import pallas as pl
from jax.experimental.pallas import tpu as pltpu
from jax.experimental.pallas import tpu_sc as plsc

_N = 10000
_E = 160000
_H = 512
_G = 64
_C = 10
_F = 128
_LANES = 16
_NC = 2
_NS = 16
_BLK = 80
_EPT = _E // _NS
_NBLK = _EPT // _BLK
_RPT = 624
_RTAIL = _N - _RPT * _NS


def _edge_stage(tabs, src, dst, w, we_sl, be_sl):
    P = len(tabs)
    PH = P // _NC
    mesh = plsc.VectorSubcoreMesh(
        core_axis_name="c", subcore_axis_name="s",
        num_cores=_NC, num_subcores=_NS)
    out_type = tuple(
        jax.ShapeDtypeStruct((_N, _F), jnp.float32) for _ in range(P))

    @functools.partial(
        pl.kernel, out_type=out_type, mesh=mesh,
        scratch_types=[
            pltpu.VMEM((_BLK,), jnp.int32),
            pltpu.VMEM((_BLK,), jnp.int32),
            pltpu.VMEM((_BLK,), jnp.float32),
            pltpu.VMEM((_BLK, _F), jnp.float32),
            pltpu.VMEM((_F,), jnp.float32),
            pltpu.VMEM((_F,), jnp.float32),
            pltpu.VMEM_SHARED((_N, _F), jnp.float32),
            pltpu.SemaphoreType.DMA,
        ])
    def ker(*args):
        tab_refs = args[:P]
        src_r, dst_r, w_r, we_r, be_r = args[P:P + 5]
        out_refs = args[P + 5:P + 5 + P]
        idx_v, dst_v, w_v, rows_v, we_v, be_v, aggr, sem = args[P + 5 + P:]
        c = lax.axis_index("c")
        s = lax.axis_index("s")
        r0 = s * _RPT
        ebase = s * _EPT

        def do_pass(p):
            tab = tab_refs[p]
            out = out_refs[p]
            pltpu.sync_copy(tab.at[pl.ds(r0, _RPT)], aggr.at[pl.ds(r0, _RPT)])

            @pl.when(s == _NS - 1)
            def _():
                pltpu.sync_copy(tab.at[pl.ds(_RPT * _NS, _RTAIL)],
                                aggr.at[pl.ds(_RPT * _NS, _RTAIL)])

            pltpu.sync_copy(we_r.at[p], we_v)
            pltpu.sync_copy(be_r.at[p], be_v)
            plsc.subcore_barrier()

            def blk_body(b, carry):
                base = ebase + b * _BLK
                pltpu.sync_copy(src_r.at[pl.ds(base, _BLK)], idx_v)
                pltpu.sync_copy(dst_r.at[pl.ds(base, _BLK)], dst_v)
                pltpu.sync_copy(w_r.at[pl.ds(base, _BLK)], w_v)
                pltpu.async_copy(tab.at[idx_v], rows_v, sem).wait()

                def grp_body(g, carry2):
                    wv16 = w_v[pl.ds(g * _LANES, _LANES)]
                    for l in range(_LANES):
                        e = g * _LANES + l
                        wsc = wv16[l]
                        for j in range(_F // _LANES):
                            sl = pl.ds(j * _LANES, _LANES)
                            row = rows_v[e, sl]
                            m = jnp.maximum(
                                row + (wsc * we_v[sl] + be_v[sl]), 0.0)
                            rows_v[e, sl] = m
                    return carry2

                lax.fori_loop(0, _BLK // _LANES, grp_body, 0)
                pltpu.sync_copy(rows_v, aggr.at[dst_v], add=True)
                return carry

            lax.fori_loop(0, _NBLK, blk_body, 0)
            plsc.subcore_barrier()
            pltpu.sync_copy(aggr.at[pl.ds(r0, _RPT)], out.at[pl.ds(r0, _RPT)])

            @pl.when(s == _NS - 1)
            def _():
                pltpu.sync_copy(aggr.at[pl.ds(_RPT * _NS, _RTAIL)],
                                out.at[pl.ds(_RPT * _NS, _RTAIL)])

            plsc.subcore_barrier()

        for ci in range(_NC):
            @pl.when(c == ci)
            def _():
                for q in range(PH):
                    do_pass(ci * PH + q)

    return ker(*tabs, src, dst, w, we_sl, be_sl)


def _mlp_stage(slices, W1f, b1f, W2f, b2f):
    P_in = len(slices)
    d_in = P_in * _F
    bn = 2000
    grid = (_N // bn,)
    P_out = _H // _F

    def body(*refs):
        x = jnp.concatenate([r[...] for r in refs[:P_in]], axis=1)
        W1 = refs[P_in][...]
        b1 = refs[P_in + 1][...]
        W2 = refs[P_in + 2][...]
        b2 = refs[P_in + 3][...]
        out_refs = refs[P_in + 4:]
        h1 = jnp.maximum(
            jnp.dot(x, W1, preferred_element_type=jnp.float32) + b1, 0.0)
        h2 = jnp.maximum(
            jnp.dot(h1, W2, preferred_element_type=jnp.float32) + b2, 0.0)
        for q in range(P_out):
            out_refs[q][...] = h2[:, q * _F:(q + 1) * _F]

    in_specs = (
        [pl.BlockSpec((bn, _F), lambda i: (i, 0)) for _ in range(P_in)] + [
            pl.BlockSpec((d_in, _H), lambda i: (0, 0)),
            pl.BlockSpec((1, _H), lambda i: (0, 0)),
            pl.BlockSpec((_H, _H), lambda i: (0, 0)),
            pl.BlockSpec((1, _H), lambda i: (0, 0)),
        ])
    out_specs = [pl.BlockSpec((bn, _F), lambda i: (i, 0))
                 for _ in range(P_out)]
    out_shape = [jax.ShapeDtypeStruct((_N, _F), jnp.float32)
                 for _ in range(P_out)]
    return pl.pallas_call(
        body, grid=grid, in_specs=in_specs, out_specs=out_specs,
        out_shape=out_shape)(*slices, W1f, b1f, W2f, b2f)


def _pool_stage(slices, batch3, Wl, bl):
    P_in = len(slices)
    bn = 2000
    grid = (_N // bn,)

    def body(*refs):
        i = pl.program_id(0)
        x = jnp.concatenate([r[...] for r in refs[:P_in]], axis=1)
        b = refs[P_in][0, 0, :]
        Wl_ = refs[P_in + 1][...]
        bl_ = refs[P_in + 2][...]
        out_ref = refs[P_in + 3]
        acc = refs[P_in + 4]

        @pl.when(i == 0)
        def _():
            acc[...] = jnp.zeros_like(acc)

        seg_ids = lax.broadcasted_iota(jnp.int32, (_G, bn), 0)
        seg = (seg_ids == b[None, :]).astype(jnp.float32)
        acc[...] += jnp.dot(seg, x, preferred_element_type=jnp.float32)

        @pl.when(i == grid[0] - 1)
        def _():
            out_ref[...] = jnp.dot(
                acc[...], Wl_, preferred_element_type=jnp.float32) + bl_

    in_specs = (
        [pl.BlockSpec((bn, _F), lambda i: (i, 0)) for _ in range(P_in)] + [
            pl.BlockSpec((1, 1, bn), lambda i: (i, 0, 0)),
            pl.BlockSpec((_H, _C), lambda i: (0, 0)),
            pl.BlockSpec((1, _C), lambda i: (0, 0)),
        ])
    return pl.pallas_call(
        body, grid=grid, in_specs=in_specs,
        out_specs=pl.BlockSpec((_G, _C), lambda i: (0, 0)),
        out_shape=jax.ShapeDtypeStruct((_G, _C), jnp.float32),
        scratch_shapes=[pltpu.VMEM((_G, _H), jnp.float32)],
    )(*slices, batch3, Wl, bl)


def kernel(x, edge_index, batch, edge_weight, params):
    src = edge_index[0]
    dst = edge_index[1]
    batch3 = batch.reshape(_N // 2000, 1, 2000)
    inv = 1.0 / jnp.sqrt(jnp.float32(1.0 + 1e-5))

    h_slices = [x[:, i * _F:(i + 1) * _F] for i in range(x.shape[1] // _F)]
    for name in ("conv1", "conv2", "conv3", "conv4"):
        p = params[name]
        P = len(h_slices)
        we_sl = p["We"].reshape(P, _F)
        be_sl = p["be"].reshape(P, _F)
        s1 = p["g1"] * inv
        W1f = p["W1"] * s1[None, :]
        b1f = (p["b1"] * s1 + p["bn_b1"])[None, :]
        s2 = p["g2"] * inv
        W2f = p["W2"] * s2[None, :]
        b2f = (p["b2"] * s2 + p["bn_b2"])[None, :]
        hpre = _edge_stage(h_slices, src, dst, edge_weight, we_sl, be_sl)
        h_slices = _mlp_stage(list(hpre), W1f, b1f, W2f, b2f)

    return _pool_stage(h_slices, batch3, params["Wl"], params["bl"][None, :])


# speedup vs baseline: 2.2301x; 2.1588x over previous
import functools

import jax
import jax.numpy as jnp
from jax import lax
from jax.experimental import pallas as pl
from jax.experimental.pallas import tpu as pltpu
from jax.experimental.pallas import tpu_sc as plsc

_N = 10000
_E = 160000
_H = 512
_G = 64
_C = 10
_F = 128
_LANES = 16
_NC = 2
_NS = 16
_BLK = 80
_EPT = _E // _NS
_NBLK = _EPT // _BLK
_RPT = 624
_RTAIL = _N - _RPT * _NS


def _edge_stage(tabs, src, dst, w, we_sl, be_sl):
    P = len(tabs)
    PH = P // _NC
    mesh = plsc.VectorSubcoreMesh(
        core_axis_name="c", subcore_axis_name="s",
        num_cores=_NC, num_subcores=_NS)
    out_type = tuple(
        jax.ShapeDtypeStruct((_N, _F), jnp.float32) for _ in range(P))

    @functools.partial(
        pl.kernel, out_type=out_type, mesh=mesh,
        scratch_types=[
            pltpu.VMEM((_BLK,), jnp.int32),
            pltpu.VMEM((_BLK,), jnp.int32),
            pltpu.VMEM((_BLK,), jnp.float32),
            pltpu.VMEM((_BLK, _F), jnp.float32),
            pltpu.VMEM((_BLK, _F), jnp.float32),
            pltpu.VMEM((_F,), jnp.float32),
            pltpu.VMEM((_F,), jnp.float32),
            pltpu.VMEM_SHARED((_N, _F), jnp.float32),
            pltpu.SemaphoreType.DMA,
        ])
    def ker(*args):
        tab_refs = args[:P]
        src_r, dst_r, w_r, we_r, be_r = args[P:P + 5]
        out_refs = args[P + 5:P + 5 + P]
        (idx_v, dst_v, w_v, rows_v, msg_v, we_v, be_v, aggr,
         sem) = args[P + 5 + P:]
        c = lax.axis_index("c")
        s = lax.axis_index("s")
        r0 = s * _RPT
        ebase = s * _EPT

        def do_pass(p):
            tab = tab_refs[p]
            out = out_refs[p]
            pltpu.sync_copy(tab.at[pl.ds(r0, _RPT)], aggr.at[pl.ds(r0, _RPT)])

            @pl.when(s == _NS - 1)
            def _():
                pltpu.sync_copy(tab.at[pl.ds(_RPT * _NS, _RTAIL)],
                                aggr.at[pl.ds(_RPT * _NS, _RTAIL)])

            pltpu.sync_copy(we_r.at[p], we_v)
            pltpu.sync_copy(be_r.at[p], be_v)
            plsc.subcore_barrier()
            wes = [we_v[pl.ds(j * _LANES, _LANES)]
                   for j in range(_F // _LANES)]
            bes = [be_v[pl.ds(j * _LANES, _LANES)]
                   for j in range(_F // _LANES)]

            def blk_body(b, carry):
                base = ebase + b * _BLK
                pltpu.sync_copy(src_r.at[pl.ds(base, _BLK)], idx_v)
                pltpu.sync_copy(dst_r.at[pl.ds(base, _BLK)], dst_v)
                pltpu.sync_copy(w_r.at[pl.ds(base, _BLK)], w_v)
                pltpu.async_copy(tab.at[idx_v], rows_v, sem).wait()

                def grp_body(g, carry2):
                    wv16 = w_v[pl.ds(g * _LANES, _LANES)]
                    for l in range(_LANES):
                        e = g * _LANES + l
                        wsc = wv16[l]
                        for j in range(_F // _LANES):
                            sl = pl.ds(j * _LANES, _LANES)
                            m = jnp.maximum(
                                rows_v[e, sl] + (wsc * wes[j] + bes[j]), 0.0)
                            msg_v[e, sl] = m
                    return carry2

                lax.fori_loop(0, _BLK // _LANES, grp_body, 0)
                pltpu.sync_copy(msg_v, aggr.at[dst_v], add=True)
                return carry

            lax.fori_loop(0, _NBLK, blk_body, 0)
            plsc.subcore_barrier()
            pltpu.sync_copy(aggr.at[pl.ds(r0, _RPT)], out.at[pl.ds(r0, _RPT)])

            @pl.when(s == _NS - 1)
            def _():
                pltpu.sync_copy(aggr.at[pl.ds(_RPT * _NS, _RTAIL)],
                                out.at[pl.ds(_RPT * _NS, _RTAIL)])

            plsc.subcore_barrier()

        for ci in range(_NC):
            @pl.when(c == ci)
            def _():
                for q in range(PH):
                    do_pass(ci * PH + q)

    return ker(*tabs, src, dst, w, we_sl, be_sl)


def _mlp_stage(slices, W1f, b1f, W2f, b2f):
    P_in = len(slices)
    d_in = P_in * _F
    bn = 2000
    grid = (_N // bn,)
    P_out = _H // _F

    def body(*refs):
        x = jnp.concatenate([r[...] for r in refs[:P_in]], axis=1)
        W1 = refs[P_in][...]
        b1 = refs[P_in + 1][...]
        W2 = refs[P_in + 2][...]
        b2 = refs[P_in + 3][...]
        out_refs = refs[P_in + 4:]
        h1 = jnp.maximum(
            jnp.dot(x, W1, preferred_element_type=jnp.float32) + b1, 0.0)
        h2 = jnp.maximum(
            jnp.dot(h1, W2, preferred_element_type=jnp.float32) + b2, 0.0)
        for q in range(P_out):
            out_refs[q][...] = h2[:, q * _F:(q + 1) * _F]

    in_specs = (
        [pl.BlockSpec((bn, _F), lambda i: (i, 0)) for _ in range(P_in)] + [
            pl.BlockSpec((d_in, _H), lambda i: (0, 0)),
            pl.BlockSpec((1, _H), lambda i: (0, 0)),
            pl.BlockSpec((_H, _H), lambda i: (0, 0)),
            pl.BlockSpec((1, _H), lambda i: (0, 0)),
        ])
    out_specs = [pl.BlockSpec((bn, _F), lambda i: (i, 0))
                 for _ in range(P_out)]
    out_shape = [jax.ShapeDtypeStruct((_N, _F), jnp.float32)
                 for _ in range(P_out)]
    return pl.pallas_call(
        body, grid=grid, in_specs=in_specs, out_specs=out_specs,
        out_shape=out_shape)(*slices, W1f, b1f, W2f, b2f)


def _pool_stage(slices, batch3, Wl, bl):
    P_in = len(slices)
    bn = 2000
    grid = (_N // bn,)

    def body(*refs):
        i = pl.program_id(0)
        x = jnp.concatenate([r[...] for r in refs[:P_in]], axis=1)
        b = refs[P_in][0, 0, :]
        Wl_ = refs[P_in + 1][...]
        bl_ = refs[P_in + 2][...]
        out_ref = refs[P_in + 3]
        acc = refs[P_in + 4]

        @pl.when(i == 0)
        def _():
            acc[...] = jnp.zeros_like(acc)

        seg_ids = lax.broadcasted_iota(jnp.int32, (_G, bn), 0)
        seg = (seg_ids == b[None, :]).astype(jnp.float32)
        acc[...] += jnp.dot(seg, x, preferred_element_type=jnp.float32)

        @pl.when(i == grid[0] - 1)
        def _():
            out_ref[...] = jnp.dot(
                acc[...], Wl_, preferred_element_type=jnp.float32) + bl_

    in_specs = (
        [pl.BlockSpec((bn, _F), lambda i: (i, 0)) for _ in range(P_in)] + [
            pl.BlockSpec((1, 1, bn), lambda i: (i, 0, 0)),
            pl.BlockSpec((_H, _C), lambda i: (0, 0)),
            pl.BlockSpec((1, _C), lambda i: (0, 0)),
        ])
    return pl.pallas_call(
        body, grid=grid, in_specs=in_specs,
        out_specs=pl.BlockSpec((_G, _C), lambda i: (0, 0)),
        out_shape=jax.ShapeDtypeStruct((_G, _C), jnp.float32),
        scratch_shapes=[pltpu.VMEM((_G, _H), jnp.float32)],
    )(*slices, batch3, Wl, bl)


def kernel(x, edge_index, batch, edge_weight, params):
    src = edge_index[0]
    dst = edge_index[1]
    batch3 = batch.reshape(_N // 2000, 1, 2000)
    inv = 1.0 / jnp.sqrt(jnp.float32(1.0 + 1e-5))

    h_slices = [x[:, i * _F:(i + 1) * _F] for i in range(x.shape[1] // _F)]
    for name in ("conv1", "conv2", "conv3", "conv4"):
        p = params[name]
        P = len(h_slices)
        we_sl = p["We"].reshape(P, _F)
        be_sl = p["be"].reshape(P, _F)
        s1 = p["g1"] * inv
        W1f = p["W1"] * s1[None, :]
        b1f = (p["b1"] * s1 + p["bn_b1"])[None, :]
        s2 = p["g2"] * inv
        W2f = p["W2"] * s2[None, :]
        b2f = (p["b2"] * s2 + p["bn_b2"])[None, :]
        hpre = _edge_stage(h_slices, src, dst, edge_weight, we_sl, be_sl)
        h_slices = _mlp_stage(list(hpre), W1f, b1f, W2f, b2f)

    return _pool_stage(h_slices, batch3, params["Wl"], params["bl"][None, :])


# speedup vs baseline: 5.1376x; 2.3038x over previous
import functools

import jax
import jax.numpy as jnp
from jax import lax
from jax.experimental import pallas as pl
from jax.experimental.pallas import tpu as pltpu
from jax.experimental.pallas import tpu_sc as plsc

_N = 10000
_E = 160000
_H = 512
_G = 64
_C = 10
_F = 128
_LANES = 16
_NC = 2
_NS = 16
_BLK = 80
_EPT = _E // _NS
_NBLK = _EPT // _BLK
_RPT = 624
_RTAIL = _N - _RPT * _NS


def _interleave_bf16(h):
    shp = h.shape
    k = shp[-1]
    h4 = h.reshape(*shp[:-1], k // 32, 2, _LANES)
    h4 = jnp.swapaxes(h4, -1, -2)
    return h4.reshape(*shp).astype(jnp.bfloat16)


def _edge_stage(tabs, epk, w, we_sl, be_sl):
    P = tabs.shape[0]
    PH = P // _NC
    mesh = plsc.VectorSubcoreMesh(
        core_axis_name="c", subcore_axis_name="s",
        num_cores=_NC, num_subcores=_NS)
    out_type = jax.ShapeDtypeStruct((P, _N, _F), jnp.float32)

    scratch = []
    for _ in range(4):
        scratch += [pltpu.VMEM((2, _BLK), jnp.int32),
                    pltpu.VMEM((_BLK,), jnp.float32),
                    pltpu.SemaphoreType.DMA]
    for _ in range(2):
        scratch += [pltpu.VMEM((_BLK, _F), jnp.float32),
                    pltpu.VMEM((_BLK, _F), jnp.float32),
                    pltpu.SemaphoreType.DMA,
                    pltpu.SemaphoreType.DMA]
    scratch += [pltpu.VMEM((_F,), jnp.float32),
                pltpu.VMEM((_F,), jnp.float32),
                pltpu.VMEM_SHARED((_N, _F), jnp.float32)]

    @functools.partial(
        pl.kernel, out_type=out_type, mesh=mesh, scratch_types=scratch)
    def ker(*args):
        tab_r, epk_r, w_r, we_r, be_r, out_r = args[:6]
        rest = args[6:]
        mbufs = [rest[3 * m:3 * m + 3] for m in range(4)]
        vbufs = [rest[12 + 4 * v:12 + 4 * v + 4] for v in range(2)]
        we_v, be_v, aggr = rest[20:23]
        c = lax.axis_index("c")
        s = lax.axis_index("s")
        r0 = s * _RPT
        ebase = s * _EPT

        def do_pass(p):
            tab = tab_r.at[p]
            out = out_r.at[p]

            def metas(b, mb, sync=False):
                mpk, wb, semm = mbufs[mb]
                gi = s * _NBLK + b
                base = ebase + b * _BLK
                if sync:
                    pltpu.sync_copy(epk_r.at[gi], mpk)
                    pltpu.sync_copy(w_r.at[pl.ds(base, _BLK)], wb)
                else:
                    pltpu.async_copy(epk_r.at[gi], mpk, semm)
                    pltpu.async_copy(w_r.at[pl.ds(base, _BLK)], wb, semm)

            def wait_metas(mb):
                mpk, wb, semm = mbufs[mb]
                pltpu.make_async_copy(epk_r.at[s * _NBLK], mpk, semm).wait()
                pltpu.make_async_copy(w_r.at[pl.ds(ebase, _BLK)], wb,
                                      semm).wait()

            def gather(mb, pv):
                mpk = mbufs[mb][0]
                rows, _, semg, _ = vbufs[pv]
                pltpu.async_copy(tab.at[mpk.at[0]], rows, semg)

            def wait_gather(mb, pv):
                mpk = mbufs[mb][0]
                rows, _, semg, _ = vbufs[pv]
                pltpu.make_async_copy(tab.at[mpk.at[0]], rows, semg).wait()

            def compute_group(wb, rows, msg, g):
                wv16 = wb[pl.ds(g * _LANES, _LANES)]
                for l in range(_LANES):
                    e = g * _LANES + l
                    wsc = wv16[l]
                    for j in range(_F // _LANES):
                        sl = pl.ds(j * _LANES, _LANES)
                        m = jnp.maximum(
                            rows[e, sl] + (wsc * wes[j] + bes[j]), 0.0)
                        msg[e, sl] = m

            def compute(mb, pv):
                wb = mbufs[mb][1]
                rows, msg = vbufs[pv][0:2]

                def grp_body(g, carry2):
                    compute_group(wb, rows, msg, g)
                    return carry2

                lax.fori_loop(0, _BLK // _LANES, grp_body, 0)

            def scatter(mb, pv):
                mpk = mbufs[mb][0]
                _, msg, _, sems = vbufs[pv]
                pltpu.async_copy(msg, aggr.at[mpk.at[1]], sems, add=True)

            def wait_scatter(mb, pv):
                mpk = mbufs[mb][0]
                _, msg, _, sems = vbufs[pv]
                pltpu.make_async_copy(msg, aggr.at[mpk.at[1]], sems).wait()

            pltpu.sync_copy(tab.at[pl.ds(r0, _RPT)], aggr.at[pl.ds(r0, _RPT)])

            @pl.when(s == _NS - 1)
            def _():
                pltpu.sync_copy(tab.at[pl.ds(_RPT * _NS, _RTAIL)],
                                aggr.at[pl.ds(_RPT * _NS, _RTAIL)])

            pltpu.sync_copy(we_r.at[p], we_v)
            pltpu.sync_copy(be_r.at[p], be_v)
            plsc.subcore_barrier()
            wes = [we_v[pl.ds(j * _LANES, _LANES)]
                   for j in range(_F // _LANES)]
            bes = [be_v[pl.ds(j * _LANES, _LANES)]
                   for j in range(_F // _LANES)]

            metas(0, 0, sync=True)
            gather(0, 0)
            metas(1, 1)

            def quad_body(i, carry):
                for k in range(4):
                    b = 4 * i + k
                    mb, pv = k, k % 2

                    wait_gather(mb, pv)
                    if k < 2:
                        @pl.when(i > 0)
                        def _():
                            wait_scatter((k + 2) % 4, pv)
                    else:
                        wait_scatter((k + 2) % 4, pv)

                    @pl.when(b + 2 < _NBLK)
                    def _():
                        metas(b + 2, (k + 2) % 4)

                    wait_metas((k + 1) % 4)
                    gather((k + 1) % 4, (k + 1) % 2)
                    compute(mb, pv)
                    scatter(mb, pv)
                return carry

            lax.fori_loop(0, (_NBLK - 1) // 4, quad_body, 0)
            wait_gather(0, 0)
            wait_scatter(2, 0)
            compute(0, 0)
            scatter(0, 0)
            wait_scatter(3, 1)
            wait_scatter(0, 0)
            plsc.subcore_barrier()
            pltpu.sync_copy(aggr.at[pl.ds(r0, _RPT)], out.at[pl.ds(r0, _RPT)])

            @pl.when(s == _NS - 1)
            def _():
                pltpu.sync_copy(aggr.at[pl.ds(_RPT * _NS, _RTAIL)],
                                out.at[pl.ds(_RPT * _NS, _RTAIL)])

            plsc.subcore_barrier()

        def pass_body(q, carry):
            do_pass(c * PH + q)
            return carry

        lax.fori_loop(0, PH, pass_body, 0)

    return ker(tabs, epk, w, we_sl, be_sl)


def _mlp_stage(hs, W1f, b1f, W2f, b2f):
    P_in = hs.shape[0]
    d_in = P_in * _F
    bn = 2000
    grid = (_N // bn,)
    P_out = _H // _F

    def body(x_ref, W1_ref, b1_ref, W2_ref, b2_ref, out_ref):
        x = jnp.concatenate([x_ref[q] for q in range(P_in)], axis=1)
        h1 = jnp.maximum(
            jnp.dot(x, W1_ref[...], preferred_element_type=jnp.float32)
            + b1_ref[...], 0.0)
        h2 = jnp.maximum(
            jnp.dot(h1, W2_ref[...], preferred_element_type=jnp.float32)
            + b2_ref[...], 0.0)
        for q in range(P_out):
            out_ref[q, :, :] = h2[:, q * _F:(q + 1) * _F]

    in_specs = [
        pl.BlockSpec((P_in, bn, _F), lambda i: (0, i, 0)),
        pl.BlockSpec((d_in, _H), lambda i: (0, 0)),
        pl.BlockSpec((1, _H), lambda i: (0, 0)),
        pl.BlockSpec((_H, _H), lambda i: (0, 0)),
        pl.BlockSpec((1, _H), lambda i: (0, 0)),
    ]
    return pl.pallas_call(
        body, grid=grid, in_specs=in_specs,
        out_specs=pl.BlockSpec((P_out, bn, _F), lambda i: (0, i, 0)),
        out_shape=jax.ShapeDtypeStruct((P_out, _N, _F), jnp.float32),
    )(hs, W1f, b1f, W2f, b2f)


def _pool_stage(hs, batch3, Wl, bl):
    P_in = hs.shape[0]
    bn = 2000
    grid = (_N // bn,)

    def body(x_ref, b_ref, Wl_ref, bl_ref, out_ref, acc):
        i = pl.program_id(0)
        x = jnp.concatenate([x_ref[q] for q in range(P_in)], axis=1)
        b = b_ref[0, 0, :]

        @pl.when(i == 0)
        def _():
            acc[...] = jnp.zeros_like(acc)

        seg_ids = lax.broadcasted_iota(jnp.int32, (_G, bn), 0)
        seg = (seg_ids == b[None, :]).astype(jnp.float32)
        acc[...] += jnp.dot(seg, x, preferred_element_type=jnp.float32)

        @pl.when(i == grid[0] - 1)
        def _():
            out_ref[...] = jnp.dot(
                acc[...], Wl_ref[...],
                preferred_element_type=jnp.float32) + bl_ref[...]

    in_specs = [
        pl.BlockSpec((P_in, bn, _F), lambda i: (0, i, 0)),
        pl.BlockSpec((1, 1, bn), lambda i: (i, 0, 0)),
        pl.BlockSpec((_H, _C), lambda i: (0, 0)),
        pl.BlockSpec((1, _C), lambda i: (0, 0)),
    ]
    return pl.pallas_call(
        body, grid=grid, in_specs=in_specs,
        out_specs=pl.BlockSpec((_G, _C), lambda i: (0, 0)),
        out_shape=jax.ShapeDtypeStruct((_G, _C), jnp.float32),
        scratch_shapes=[pltpu.VMEM((_G, _H), jnp.float32)],
    )(hs, batch3, Wl, bl)


def kernel(x, edge_index, batch, edge_weight, params):
    epk = jnp.concatenate(
        [edge_index[0][None], edge_index[1][None]], axis=0)
    epk = epk.reshape(2, _E // _BLK, _BLK).transpose(1, 0, 2)
    batch3 = batch.reshape(_N // 2000, 1, 2000)
    inv = 1.0 / jnp.sqrt(jnp.float32(1.0 + 1e-5))

    h = x.reshape(_N, x.shape[1] // _F, _F).transpose(1, 0, 2)
    for name in ("conv1", "conv2", "conv3", "conv4"):
        p = params[name]
        P = h.shape[0]
        we_sl = p["We"].reshape(P, _F)
        be_sl = p["be"].reshape(P, _F)
        s1 = p["g1"] * inv
        W1f = p["W1"] * s1[None, :]
        b1f = (p["b1"] * s1 + p["bn_b1"])[None, :]
        s2 = p["g2"] * inv
        W2f = p["W2"] * s2[None, :]
        b2f = (p["b2"] * s2 + p["bn_b2"])[None, :]
        hpre = _edge_stage(h, epk, edge_weight, we_sl, be_sl)
        h = _mlp_stage(hpre, W1f, b1f, W2f, b2f)

    return _pool_stage(h, batch3, params["Wl"], params["bl"][None, :])


# speedup vs baseline: 5.1474x; 1.0019x over previous
import functools

import jax
import jax.numpy as jnp
from jax import lax
from jax.experimental import pallas as pl
from jax.experimental.pallas import tpu as pltpu
from jax.experimental.pallas import tpu_sc as plsc

_N = 10000
_E = 160000
_H = 512
_G = 64
_C = 10
_F = 128
_LANES = 16
_NC = 2
_NS = 16
_BLK = 80
_EPT = _E // _NS
_NBLK = _EPT // _BLK
_RPT = 624
_RTAIL = _N - _RPT * _NS


def _interleave_bf16(h):
    shp = h.shape
    k = shp[-1]
    h4 = h.reshape(*shp[:-1], k // 32, 2, _LANES)
    h4 = jnp.swapaxes(h4, -1, -2)
    return h4.reshape(*shp).astype(jnp.bfloat16)


def _edge_stage(tabs, epk, w, we_sl, be_sl):
    P = tabs.shape[0]
    PH = P // _NC
    mesh = plsc.VectorSubcoreMesh(
        core_axis_name="c", subcore_axis_name="s",
        num_cores=_NC, num_subcores=_NS)
    out_type = jax.ShapeDtypeStruct((P, _N, _F), jnp.float32)

    scratch = []
    for _ in range(4):
        scratch += [pltpu.VMEM((2, _BLK), jnp.int32),
                    pltpu.VMEM((_BLK,), jnp.float32),
                    pltpu.SemaphoreType.DMA]
    for _ in range(2):
        scratch += [pltpu.VMEM((_BLK, _F), jnp.float32),
                    pltpu.VMEM((_BLK, _F), jnp.float32),
                    pltpu.SemaphoreType.DMA,
                    pltpu.SemaphoreType.DMA]
    scratch += [pltpu.VMEM((_F,), jnp.float32),
                pltpu.VMEM((_F,), jnp.float32),
                pltpu.VMEM_SHARED((_N, _F), jnp.float32)]

    @functools.partial(
        pl.kernel, out_type=out_type, mesh=mesh, scratch_types=scratch)
    def ker(*args):
        tab_r, epk_r, w_r, we_r, be_r, out_r = args[:6]
        rest = args[6:]
        mbufs = [rest[3 * m:3 * m + 3] for m in range(4)]
        vbufs = [rest[12 + 4 * v:12 + 4 * v + 4] for v in range(2)]
        we_v, be_v, aggr = rest[20:23]
        c = lax.axis_index("c")
        s = lax.axis_index("s")
        r0 = s * _RPT
        ebase = s * _EPT

        def do_pass(p):
            tab = tab_r.at[p]
            out = out_r.at[p]

            def metas(b, mb, sync=False):
                mpk, wb, semm = mbufs[mb]
                gi = s * _NBLK + b
                base = ebase + b * _BLK
                if sync:
                    pltpu.sync_copy(epk_r.at[gi], mpk)
                    pltpu.sync_copy(w_r.at[pl.ds(base, _BLK)], wb)
                else:
                    pltpu.async_copy(epk_r.at[gi], mpk, semm)
                    pltpu.async_copy(w_r.at[pl.ds(base, _BLK)], wb, semm)

            def wait_metas(mb):
                mpk, wb, semm = mbufs[mb]
                pltpu.make_async_copy(epk_r.at[s * _NBLK], mpk, semm).wait()
                pltpu.make_async_copy(w_r.at[pl.ds(ebase, _BLK)], wb,
                                      semm).wait()

            def gather(mb, pv):
                mpk = mbufs[mb][0]
                rows, _, semg, _ = vbufs[pv]
                pltpu.async_copy(tab.at[mpk.at[0]], rows, semg)

            def wait_gather(mb, pv):
                mpk = mbufs[mb][0]
                rows, _, semg, _ = vbufs[pv]
                pltpu.make_async_copy(tab.at[mpk.at[0]], rows, semg).wait()

            def compute_group(wb, rows, msg, g):
                wv16 = wb[pl.ds(g * _LANES, _LANES)]
                for l in range(_LANES):
                    e = g * _LANES + l
                    wsc = wv16[l]
                    for j in range(_F // _LANES):
                        sl = pl.ds(j * _LANES, _LANES)
                        m = jnp.maximum(
                            rows[e, sl] + (wsc * wes[j] + bes[j]), 0.0)
                        msg[e, sl] = m

            def compute(mb, pv):
                wb = mbufs[mb][1]
                rows, msg = vbufs[pv][0:2]

                def grp_body(g, carry2):
                    compute_group(wb, rows, msg, g)
                    return carry2

                lax.fori_loop(0, _BLK // _LANES, grp_body, 0)

            def scatter(mb, pv):
                mpk = mbufs[mb][0]
                _, msg, _, sems = vbufs[pv]
                pltpu.async_copy(msg, aggr.at[mpk.at[1]], sems, add=True)

            def wait_scatter(mb, pv):
                mpk = mbufs[mb][0]
                _, msg, _, sems = vbufs[pv]
                pltpu.make_async_copy(msg, aggr.at[mpk.at[1]], sems).wait()

            pltpu.sync_copy(tab.at[pl.ds(r0, _RPT)], aggr.at[pl.ds(r0, _RPT)])

            @pl.when(s == _NS - 1)
            def _():
                pltpu.sync_copy(tab.at[pl.ds(_RPT * _NS, _RTAIL)],
                                aggr.at[pl.ds(_RPT * _NS, _RTAIL)])

            pltpu.sync_copy(we_r.at[p], we_v)
            pltpu.sync_copy(be_r.at[p], be_v)
            plsc.subcore_barrier()
            wes = [we_v[pl.ds(j * _LANES, _LANES)]
                   for j in range(_F // _LANES)]
            bes = [be_v[pl.ds(j * _LANES, _LANES)]
                   for j in range(_F // _LANES)]

            metas(0, 0, sync=True)
            gather(0, 0)
            metas(1, 1)

            def quad_body(i, carry):
                for k in range(4):
                    b = 4 * i + k
                    mb, pv = k, k % 2

                    wait_gather(mb, pv)
                    wait_metas((k + 1) % 4)
                    gather((k + 1) % 4, (k + 1) % 2)
                    if k < 2:
                        @pl.when(i > 0)
                        def _():
                            wait_scatter((k + 2) % 4, pv)
                    else:
                        wait_scatter((k + 2) % 4, pv)

                    @pl.when(b + 2 < _NBLK)
                    def _():
                        metas(b + 2, (k + 2) % 4)

                    compute(mb, pv)
                    scatter(mb, pv)
                return carry

            lax.fori_loop(0, (_NBLK - 1) // 4, quad_body, 0)
            wait_gather(0, 0)
            wait_scatter(2, 0)
            compute(0, 0)
            scatter(0, 0)
            wait_scatter(3, 1)
            wait_scatter(0, 0)
            plsc.subcore_barrier()
            pltpu.sync_copy(aggr.at[pl.ds(r0, _RPT)], out.at[pl.ds(r0, _RPT)])

            @pl.when(s == _NS - 1)
            def _():
                pltpu.sync_copy(aggr.at[pl.ds(_RPT * _NS, _RTAIL)],
                                out.at[pl.ds(_RPT * _NS, _RTAIL)])

            plsc.subcore_barrier()

        def pass_body(q, carry):
            do_pass(c * PH + q)
            return carry

        lax.fori_loop(0, PH, pass_body, 0)

    return ker(tabs, epk, w, we_sl, be_sl)


def _mlp_stage(hs, W1f, b1f, W2f, b2f):
    P_in = hs.shape[0]
    d_in = P_in * _F
    bn = 2000
    grid = (_N // bn,)
    P_out = _H // _F

    def body(x_ref, W1_ref, b1_ref, W2_ref, b2_ref, out_ref):
        x = jnp.concatenate([x_ref[q] for q in range(P_in)], axis=1)
        h1 = jnp.maximum(
            jnp.dot(x, W1_ref[...], preferred_element_type=jnp.float32)
            + b1_ref[...], 0.0)
        h2 = jnp.maximum(
            jnp.dot(h1, W2_ref[...], preferred_element_type=jnp.float32)
            + b2_ref[...], 0.0)
        for q in range(P_out):
            out_ref[q, :, :] = h2[:, q * _F:(q + 1) * _F]

    in_specs = [
        pl.BlockSpec((P_in, bn, _F), lambda i: (0, i, 0)),
        pl.BlockSpec((d_in, _H), lambda i: (0, 0)),
        pl.BlockSpec((1, _H), lambda i: (0, 0)),
        pl.BlockSpec((_H, _H), lambda i: (0, 0)),
        pl.BlockSpec((1, _H), lambda i: (0, 0)),
    ]
    return pl.pallas_call(
        body, grid=grid, in_specs=in_specs,
        out_specs=pl.BlockSpec((P_out, bn, _F), lambda i: (0, i, 0)),
        out_shape=jax.ShapeDtypeStruct((P_out, _N, _F), jnp.float32),
    )(hs, W1f, b1f, W2f, b2f)


def _pool_stage(hs, batch3, Wl, bl):
    P_in = hs.shape[0]
    bn = 2000
    grid = (_N // bn,)

    def body(x_ref, b_ref, Wl_ref, bl_ref, out_ref, acc):
        i = pl.program_id(0)
        x = jnp.concatenate([x_ref[q] for q in range(P_in)], axis=1)
        b = b_ref[0, 0, :]

        @pl.when(i == 0)
        def _():
            acc[...] = jnp.zeros_like(acc)

        seg_ids = lax.broadcasted_iota(jnp.int32, (_G, bn), 0)
        seg = (seg_ids == b[None, :]).astype(jnp.float32)
        acc[...] += jnp.dot(seg, x, preferred_element_type=jnp.float32)

        @pl.when(i == grid[0] - 1)
        def _():
            out_ref[...] = jnp.dot(
                acc[...], Wl_ref[...],
                preferred_element_type=jnp.float32) + bl_ref[...]

    in_specs = [
        pl.BlockSpec((P_in, bn, _F), lambda i: (0, i, 0)),
        pl.BlockSpec((1, 1, bn), lambda i: (i, 0, 0)),
        pl.BlockSpec((_H, _C), lambda i: (0, 0)),
        pl.BlockSpec((1, _C), lambda i: (0, 0)),
    ]
    return pl.pallas_call(
        body, grid=grid, in_specs=in_specs,
        out_specs=pl.BlockSpec((_G, _C), lambda i: (0, 0)),
        out_shape=jax.ShapeDtypeStruct((_G, _C), jnp.float32),
        scratch_shapes=[pltpu.VMEM((_G, _H), jnp.float32)],
    )(hs, batch3, Wl, bl)


def kernel(x, edge_index, batch, edge_weight, params):
    epk = jnp.concatenate(
        [edge_index[0][None], edge_index[1][None]], axis=0)
    epk = epk.reshape(2, _E // _BLK, _BLK).transpose(1, 0, 2)
    batch3 = batch.reshape(_N // 2000, 1, 2000)
    inv = 1.0 / jnp.sqrt(jnp.float32(1.0 + 1e-5))

    h = x.reshape(_N, x.shape[1] // _F, _F).transpose(1, 0, 2)
    for name in ("conv1", "conv2", "conv3", "conv4"):
        p = params[name]
        P = h.shape[0]
        we_sl = p["We"].reshape(P, _F)
        be_sl = p["be"].reshape(P, _F)
        s1 = p["g1"] * inv
        W1f = p["W1"] * s1[None, :]
        b1f = (p["b1"] * s1 + p["bn_b1"])[None, :]
        s2 = p["g2"] * inv
        W2f = p["W2"] * s2[None, :]
        b2f = (p["b2"] * s2 + p["bn_b2"])[None, :]
        hpre = _edge_stage(h, epk, edge_weight, we_sl, be_sl)
        h = _mlp_stage(hpre, W1f, b1f, W2f, b2f)

    return _pool_stage(h, batch3, params["Wl"], params["bl"][None, :])
